# baseline (speedup 1.0000x reference)
import functools

import jax
import jax.numpy as jnp
from jax import lax
from jax.experimental import pallas as pl
from jax.experimental.pallas import tpu as pltpu
from jax.experimental.pallas import tpu_sc as plsc

B = 1024
S = 200
D = 256
VG = 50000
VI = 30000
VO = 100000

S_P = 224
VG_P = 50048
_VGC = 2048
_TC_GRID = (VG_P + _VGC - 1) // _VGC

_NEG = -1e30


def _p1_body(x_ref, wg_ref, bg_ref, wp_ref, bp_ref, al_ref,
             m_ref, s_ref, itp_ref, beta_ref):
    v = pl.program_id(0)

    @pl.when(v == 0)
    def _init():
        z = jnp.dot(x_ref[...], wp_ref[...],
                    preferred_element_type=jnp.float32) + bp_ref[...]
        itp = jax.nn.sigmoid(z)
        itp_ref[...] = itp
        beta_ref[...] = (1.0 - itp) * al_ref[...]
        m_ref[...] = jnp.full(m_ref.shape, _NEG, jnp.float32)
        s_ref[...] = jnp.zeros(s_ref.shape, jnp.float32)

    logits = jnp.dot(x_ref[...], wg_ref[...],
                     preferred_element_type=jnp.float32) + bg_ref[...]
    col = v * _VGC + lax.broadcasted_iota(jnp.int32, logits.shape, 1)
    logits = jnp.where(col < VG, logits, _NEG)
    m_old = m_ref[...]
    m_new = jnp.maximum(m_old, jnp.max(logits, axis=1, keepdims=True))
    s_ref[...] = (s_ref[...] * jnp.exp(m_old - m_new)
                  + jnp.sum(jnp.exp(logits - m_new), axis=1, keepdims=True))
    m_ref[...] = m_new


_pass1 = pl.pallas_call(
    _p1_body,
    grid=(_TC_GRID,),
    in_specs=[
        pl.BlockSpec((B, D), lambda v: (0, 0)),
        pl.BlockSpec((D, _VGC), lambda v: (0, v)),
        pl.BlockSpec((1, _VGC), lambda v: (0, v)),
        pl.BlockSpec((D, 1), lambda v: (0, 0)),
        pl.BlockSpec((1, 1), lambda v: (0, 0)),
        pl.BlockSpec((B, S_P), lambda v: (0, 0)),
    ],
    out_specs=[
        pl.BlockSpec((B, 1), lambda v: (0, 0)),
        pl.BlockSpec((B, 1), lambda v: (0, 0)),
        pl.BlockSpec((B, 1), lambda v: (0, 0)),
        pl.BlockSpec((B, S_P), lambda v: (0, 0)),
    ],
    out_shape=[
        jax.ShapeDtypeStruct((B, 1), jnp.float32),
        jax.ShapeDtypeStruct((B, 1), jnp.float32),
        jax.ShapeDtypeStruct((B, 1), jnp.float32),
        jax.ShapeDtypeStruct((B, S_P), jnp.float32),
    ],
)


def _p2_body(x_ref, wg_ref, bg_ref, m_ref, s_ref, itp_ref, a_ref):
    v = pl.program_id(0)
    logits = jnp.dot(x_ref[...], wg_ref[...],
                     preferred_element_type=jnp.float32) + bg_ref[...]
    col = v * _VGC + lax.broadcasted_iota(jnp.int32, logits.shape, 1)
    logits = jnp.where(col < VG, logits, _NEG)
    coef = itp_ref[...] / s_ref[...]
    a_ref[...] = jnp.exp(logits - m_ref[...]) * coef


_pass2 = pl.pallas_call(
    _p2_body,
    grid=(_TC_GRID,),
    in_specs=[
        pl.BlockSpec((B, D), lambda v: (0, 0)),
        pl.BlockSpec((D, _VGC), lambda v: (0, v)),
        pl.BlockSpec((1, _VGC), lambda v: (0, v)),
        pl.BlockSpec((B, 1), lambda v: (0, 0)),
        pl.BlockSpec((B, 1), lambda v: (0, 0)),
        pl.BlockSpec((B, 1), lambda v: (0, 0)),
    ],
    out_specs=pl.BlockSpec((B, _VGC), lambda v: (0, v)),
    out_shape=jax.ShapeDtypeStruct((B, VG_P), jnp.float32),
)


_NC, _NS = 2, 16
_NW = _NC * _NS
_RPT = B // _NW
_CH = 4096
_CHUNKS = [(i * _CH, _CH) for i in range(VG_P // _CH)]
if VG_P % _CH:
    _CHUNKS.append(((VG_P // _CH) * _CH, VG_P % _CH))


def _sc_body(a_hbm, g_hbm, beta_hbm, ctx_hbm, i2o_hbm, out_hbm,
             row_v, val0, val1, idx0, idx1, ctxa, ctxb, ctoa, ctob, betab,
             sv0, sv1, si0, si1, sca, scb, sbe, sg0, sg1):
    c = lax.axis_index("c")
    s = lax.axis_index("s")
    base = (s * _NC + c) * _RPT

    def row_body(r, carry):
        row = base + r
        ctx0 = row * S_P
        h_ca = pltpu.async_copy(ctx_hbm.at[pl.ds(ctx0, 112)], ctxa, sca)
        h_cb = pltpu.async_copy(ctx_hbm.at[pl.ds(ctx0 + 112, 112)], ctxb, scb)
        h_be = pltpu.async_copy(beta_hbm.at[pl.ds(ctx0, S_P)], betab, sbe)

        def issue(ci):
            off, sz = _CHUNKS[ci]
            vb, ib, sv, si = ((val0, idx0, sv0, si0) if ci % 2 == 0
                              else (val1, idx1, sv1, si1))
            hv = pltpu.async_copy(a_hbm.at[pl.ds(row * VG_P + off, sz)],
                                  vb.at[pl.ds(0, sz)], sv)
            hi = pltpu.async_copy(g_hbm.at[pl.ds(off, sz)],
                                  ib.at[pl.ds(0, sz)], si)
            return hv, hi

        h = issue(0)

        def zero_step(i, acc):
            row_v[pl.ds(pl.multiple_of(i * 16, 16), 16)] = (
                jnp.zeros((16,), jnp.float32))
            return acc

        lax.fori_loop(0, VO // 16, zero_step, 0, unroll=8)

        for ci in range(len(_CHUNKS)):
            hv, hi = h
            h = issue(ci + 1) if ci + 1 < len(_CHUNKS) else None
            hv.wait()
            hi.wait()
            _, sz = _CHUNKS[ci]
            vb, ib = (val0, idx0) if ci % 2 == 0 else (val1, idx1)

            def sc_step(j, acc, vb=vb, ib=ib):
                o = pl.multiple_of(j * 16, 16)
                plsc.addupdate_scatter(row_v, [ib[pl.ds(o, 16)]],
                                       vb[pl.ds(o, 16)])
                return acc

            lax.fori_loop(0, sz // 16, sc_step, 0, unroll=8)

        h_ca.wait()
        h_cb.wait()
        h_be.wait()
        pltpu.async_copy(i2o_hbm.at[ctxa], ctoa, sg0).wait()
        pltpu.async_copy(i2o_hbm.at[ctxb], ctob, sg1).wait()
        for cto, boff in ((ctoa, 0), (ctob, 112)):

            def cs_step(j, acc, cto=cto, boff=boff):
                o = pl.multiple_of(j * 16, 16)
                plsc.addupdate_scatter(row_v, [cto[pl.ds(o, 16)]],
                                       betab[pl.ds(boff + o, 16)])
                return acc

            lax.fori_loop(0, 112 // 16, cs_step, 0)

        pltpu.sync_copy(row_v, out_hbm.at[pl.ds(row * VO, VO)])
        return carry

    lax.fori_loop(0, _RPT, row_body, 0)


@functools.cache
def _sc_scatter_kernel():
  return pl.kernel(
    _sc_body,
    out_type=jax.ShapeDtypeStruct((B * VO,), jnp.float32),
    mesh=plsc.VectorSubcoreMesh(core_axis_name="c", subcore_axis_name="s",
                                num_cores=_NC, num_subcores=_NS),
    scratch_types=[
        pltpu.VMEM((VO,), jnp.float32),
        pltpu.VMEM((_CH,), jnp.float32),
        pltpu.VMEM((_CH,), jnp.float32),
        pltpu.VMEM((_CH,), jnp.int32),
        pltpu.VMEM((_CH,), jnp.int32),
        pltpu.VMEM((112,), jnp.int32),
        pltpu.VMEM((112,), jnp.int32),
        pltpu.VMEM((112,), jnp.int32),
        pltpu.VMEM((112,), jnp.int32),
        pltpu.VMEM((S_P,), jnp.float32),
    ] + [pltpu.SemaphoreType.DMA] * 9,
    compiler_params=pltpu.CompilerParams(needs_layout_passes=False),
  )


def kernel(x, alphas, ctx_inp, W_p, b_p, W_g, b_g, gen_to_out, inp_to_out):
    xb = x.astype(jnp.bfloat16)
    wgb = W_g.astype(jnp.bfloat16)
    wpb = W_p.astype(jnp.bfloat16)
    bg2 = b_g.reshape(1, VG).astype(jnp.float32)
    bp2 = b_p.reshape(1, 1).astype(jnp.float32)
    al_p = jnp.pad(alphas, ((0, 0), (0, S_P - S)))
    ctx_p = jnp.pad(ctx_inp.astype(jnp.int32), ((0, 0), (0, S_P - S)))
    i2o = inp_to_out.astype(jnp.int32)

    order = jnp.argsort(gen_to_out.astype(jnp.int32))
    gidx = jnp.pad(gen_to_out.astype(jnp.int32)[order], (0, VG_P - VG))
    wgb = wgb[:, order]
    bg2 = bg2[:, order]

    m, sden, itp, beta = _pass1(xb, wgb, bg2, wpb, bp2, al_p)
    a = _pass2(xb, wgb, bg2, m, sden, itp)
    out = _sc_scatter_kernel()(a.reshape(-1), gidx, beta.reshape(-1),
                               ctx_p.reshape(-1), i2o)
    return out.reshape(B, VO)


# speedup vs baseline: 1.3608x; 1.3608x over previous
import functools

import jax
import jax.numpy as jnp
from jax import lax
from jax.experimental import pallas as pl
from jax.experimental.pallas import tpu as pltpu
from jax.experimental.pallas import tpu_sc as plsc

B = 1024
S = 200
D = 256
VG = 50000
VI = 30000
VO = 100000

S_P = 224
VG_PAD = 50560
_VGC = 2048
_TC_GRID = (VG_PAD + _VGC - 1) // _VGC

_NEG = -1e30


def _p1_body(x_ref, wg_ref, bg_ref, wp_ref, bp_ref, al_ref,
             m_ref, s_ref, itp_ref, beta_ref):
    v = pl.program_id(0)

    @pl.when(v == 0)
    def _init():
        z = jnp.dot(x_ref[...], wp_ref[...],
                    preferred_element_type=jnp.float32) + bp_ref[...]
        itp = jax.nn.sigmoid(z)
        itp_ref[...] = itp
        beta_ref[...] = (1.0 - itp) * al_ref[...]
        m_ref[...] = jnp.full(m_ref.shape, _NEG, jnp.float32)
        s_ref[...] = jnp.zeros(s_ref.shape, jnp.float32)

    logits = jnp.dot(x_ref[...], wg_ref[...],
                     preferred_element_type=jnp.float32) + bg_ref[...]
    col = v * _VGC + lax.broadcasted_iota(jnp.int32, logits.shape, 1)
    logits = jnp.where(col < VG, logits, _NEG)
    m_old = m_ref[...]
    m_new = jnp.maximum(m_old, jnp.max(logits, axis=1, keepdims=True))
    s_ref[...] = (s_ref[...] * jnp.exp(m_old - m_new)
                  + jnp.sum(jnp.exp(logits - m_new), axis=1, keepdims=True))
    m_ref[...] = m_new


_pass1 = pl.pallas_call(
    _p1_body,
    grid=(_TC_GRID,),
    in_specs=[
        pl.BlockSpec((B, D), lambda v: (0, 0)),
        pl.BlockSpec((D, _VGC), lambda v: (0, v)),
        pl.BlockSpec((1, _VGC), lambda v: (0, v)),
        pl.BlockSpec((D, 1), lambda v: (0, 0)),
        pl.BlockSpec((1, 1), lambda v: (0, 0)),
        pl.BlockSpec((B, S_P), lambda v: (0, 0)),
    ],
    out_specs=[
        pl.BlockSpec((B, 1), lambda v: (0, 0)),
        pl.BlockSpec((B, 1), lambda v: (0, 0)),
        pl.BlockSpec((B, 1), lambda v: (0, 0)),
        pl.BlockSpec((B, S_P), lambda v: (0, 0)),
    ],
    out_shape=[
        jax.ShapeDtypeStruct((B, 1), jnp.float32),
        jax.ShapeDtypeStruct((B, 1), jnp.float32),
        jax.ShapeDtypeStruct((B, 1), jnp.float32),
        jax.ShapeDtypeStruct((B, S_P), jnp.float32),
    ],
)


def _p2_body(x_ref, wg_ref, bg_ref, m_ref, s_ref, itp_ref, a_ref):
    v = pl.program_id(0)
    logits = jnp.dot(x_ref[...], wg_ref[...],
                     preferred_element_type=jnp.float32) + bg_ref[...]
    col = v * _VGC + lax.broadcasted_iota(jnp.int32, logits.shape, 1)
    logits = jnp.where(col < VG, logits, _NEG)
    coef = itp_ref[...] / s_ref[...]
    a_ref[...] = jnp.exp(logits - m_ref[...]) * coef


_pass2 = pl.pallas_call(
    _p2_body,
    grid=(_TC_GRID,),
    in_specs=[
        pl.BlockSpec((B, D), lambda v: (0, 0)),
        pl.BlockSpec((D, _VGC), lambda v: (0, v)),
        pl.BlockSpec((1, _VGC), lambda v: (0, v)),
        pl.BlockSpec((B, 1), lambda v: (0, 0)),
        pl.BlockSpec((B, 1), lambda v: (0, 0)),
        pl.BlockSpec((B, 1), lambda v: (0, 0)),
    ],
    out_specs=pl.BlockSpec((B, _VGC), lambda v: (0, v)),
    out_shape=jax.ShapeDtypeStruct((B, VG_PAD), jnp.float32),
)


_NC, _NS = 2, 16
_NW = _NC * _NS
_GPT = (B // 8) // _NW

_WB = 5120
_NFULL = 19
_WTAIL = VO - _NFULL * _WB
_PC = 512


def _sget(ref, idx):
    base = (idx // 16) * 16
    v = ref[pl.ds(base, 16)]
    lane = idx % 16
    sel = jnp.where(lax.broadcasted_iota(jnp.int32, (16,), 0) == lane, v, 0)
    return jnp.sum(sel)


def _sc_body(a_hbm, g_hbm, bnds_hbm, beta_hbm, ctx_hbm, i2o_hbm, out_hbm,
             s0, s1, s2, va, vb, ia, ib, ctxg, ctog, betag,
             bnds_vm, sema, semb, swb0, swb1, swb2, sctx, sbet, sgat):
    c = lax.axis_index("c")
    s = lax.axis_index("s")
    wid = s * _NC + c

    pltpu.sync_copy(bnds_hbm, bnds_vm)

    strips = (s0, s1)
    piece = ((va, ia, sema), (vb, ib, semb))
    wbsem = (swb0, swb1)

    def group_body(g, carry):
        grp = wid * _GPT + g
        g8 = pl.multiple_of(grp * 8, 8)

        coff = pl.multiple_of(g8 * S_P, 8)
        hctx = pltpu.async_copy(ctx_hbm.at[pl.ds(coff, 8 * S_P)], ctxg, sctx)
        hbeta = pltpu.async_copy(beta_hbm.at[pl.ds(coff, 8 * S_P)],
                                 betag, sbet)
        hctx.wait()
        hbeta.wait()
        gh = []
        for j in range(8 * S_P // 128):
            gh.append(pltpu.async_copy(
                i2o_hbm.at[ctxg.at[pl.ds(j * 128, 128)]],
                ctog.at[pl.ds(j * 128, 128)], sgat))
        for h in gh:
            h.wait()

        wb_pending = [None, None, None]

        for k in range(_NFULL + 1):
            tail = k == _NFULL
            slot = 2 if tail else k % 2
            strip = s2 if tail else strips[slot]
            W = _WTAIL if tail else _WB
            c0 = k * _WB
            c1 = c0 + W
            wsem = swb2 if tail else wbsem[slot]

            if wb_pending[slot] is not None:
                wb_pending[slot].wait()
                wb_pending[slot] = None

            def zrow(i, acc, strip=strip):
                o = pl.multiple_of(i * 16, 16)
                z = jnp.zeros((16,), jnp.float32)
                for rr in range(8):
                    strip[rr, pl.ds(o, 16)] = z
                return acc
            lax.fori_loop(0, W // 16, zrow, 0)

            lo = _sget(bnds_vm, k)
            hi = _sget(bnds_vm, k + 1)
            lo128 = lax.bitwise_and(lo, -128)
            n_p = lax.shift_right_logical(hi - lo128 + (_PC - 1), 9)
            n_pairs = lax.shift_right_logical(n_p + 1, 1)

            def issue(pidx, pslot, g8=g8, lo128=lo128):
                vbuf, ibuf, sem = piece[pslot]
                start = pl.multiple_of(lo128 + pidx * _PC, 128)
                pltpu.async_copy(
                    a_hbm.at[pl.ds(g8, 8), pl.ds(start, _PC)], vbuf, sem)
                pltpu.async_copy(g_hbm.at[pl.ds(start, _PC)], ibuf, sem)

            def wait_piece(pslot):
                vbuf, ibuf, sem = piece[pslot]
                pltpu.make_async_copy(
                    a_hbm.at[pl.ds(0, 8), pl.ds(0, _PC)], vbuf, sem).wait()
                pltpu.make_async_copy(
                    g_hbm.at[pl.ds(0, _PC)], ibuf, sem).wait()

            def process(pslot, strip=strip, c0=c0, c1=c1):
                vbuf, ibuf, _ = piece[pslot]

                def pv(j, acc):
                    o = pl.multiple_of(j * 16, 16)
                    iv = ibuf[pl.ds(o, 16)]
                    vcol = iv - c0
                    m = (iv >= c0) & (iv < c1)
                    for rr in range(8):
                        plsc.addupdate_scatter(
                            strip, [jnp.full((16,), rr, jnp.int32), vcol],
                            vbuf[rr, pl.ds(o, 16)], mask=m)
                    return acc
                lax.fori_loop(0, _PC // 16, pv, 0)

            @pl.when(n_p > 0)
            def _prol():
                issue(0, 0)

            def pair_body(q, acc):
                p0 = 2 * q

                @pl.when(p0 + 1 < n_p)
                def _():
                    issue(p0 + 1, 1)
                wait_piece(0)
                process(0)

                @pl.when(p0 + 2 < n_p)
                def _():
                    issue(p0 + 2, 0)

                @pl.when(p0 + 1 < n_p)
                def _():
                    wait_piece(1)
                    process(1)
                return acc

            lax.fori_loop(0, n_pairs, pair_body, 0)

            def cs(j, acc, strip=strip, c0=c0, c1=c1):
                for rr in range(8):
                    o = pl.multiple_of(rr * S_P + j * 16, 16)
                    cv = ctog[pl.ds(o, 16)]
                    m = (cv >= c0) & (cv < c1)
                    plsc.addupdate_scatter(
                        strip, [jnp.full((16,), rr, jnp.int32), cv - c0],
                        betag[pl.ds(o, 16)], mask=m)
                return acc
            lax.fori_loop(0, S_P // 16, cs, 0)

            dst = out_hbm.at[pl.ds(g8, 8), pl.ds(c0, W)]
            wb_pending[slot] = pltpu.async_copy(strip, dst, wsem)

        for slot in range(3):
            if wb_pending[slot] is not None:
                wb_pending[slot].wait()
        return carry

    lax.fori_loop(0, _GPT, group_body, 0)


@functools.cache
def _sc_scatter_kernel():
    return pl.kernel(
        _sc_body,
        out_type=jax.ShapeDtypeStruct((B, VO), jnp.float32),
        mesh=plsc.VectorSubcoreMesh(core_axis_name="c", subcore_axis_name="s",
                                    num_cores=_NC, num_subcores=_NS),
        scratch_types=[
            pltpu.VMEM((8, _WB), jnp.float32),
            pltpu.VMEM((8, _WB), jnp.float32),
            pltpu.VMEM((8, _WTAIL), jnp.float32),
            pltpu.VMEM((8, _PC), jnp.float32),
            pltpu.VMEM((8, _PC), jnp.float32),
            pltpu.VMEM((_PC,), jnp.int32),
            pltpu.VMEM((_PC,), jnp.int32),
            pltpu.VMEM((8 * S_P,), jnp.int32),
            pltpu.VMEM((8 * S_P,), jnp.int32),
            pltpu.VMEM((8 * S_P,), jnp.float32),
            pltpu.VMEM((32,), jnp.int32),
        ] + [pltpu.SemaphoreType.DMA] * 8,
        compiler_params=pltpu.CompilerParams(needs_layout_passes=False),
    )


def kernel(x, alphas, ctx_inp, W_p, b_p, W_g, b_g, gen_to_out, inp_to_out):
    xb = x.astype(jnp.bfloat16)
    wpb = W_p.astype(jnp.bfloat16)
    bp2 = b_p.reshape(1, 1).astype(jnp.float32)
    al_p = jnp.pad(alphas, ((0, 0), (0, S_P - S)))
    ctx_p = jnp.pad(ctx_inp.astype(jnp.int32), ((0, 0), (0, S_P - S)))
    i2o = inp_to_out.astype(jnp.int32)

    g32 = gen_to_out.astype(jnp.int32)
    order = jnp.argsort(g32)
    gsort = jnp.pad(g32[order], (0, VG_PAD - VG), constant_values=VO)
    wgb = W_g.astype(jnp.bfloat16)[:, order]
    bg2 = b_g.reshape(1, VG).astype(jnp.float32)[:, order]
    starts = jnp.array([k * _WB for k in range(_NFULL + 1)] + [VO],
                       dtype=jnp.int32)
    bnds = jnp.pad(jnp.searchsorted(gsort, starts).astype(jnp.int32),
                   (0, 32 - (_NFULL + 2)))

    m, sden, itp, beta = _pass1(xb, wgb, bg2, wpb, bp2, al_p)
    a = _pass2(xb, wgb, bg2, m, sden, itp)
    return _sc_scatter_kernel()(a, gsort, bnds, beta.reshape(-1),
                                ctx_p.reshape(-1), i2o)
